# Initial kernel scaffold; baseline (speedup 1.0000x reference)
import jax
import jax.numpy as jnp
from jax.experimental import pallas as pl


def kernel(h, edge_index, edge_weight, weight):
    raise NotImplementedError("write your pallas kernel here")


# speedup vs baseline: 6.8291x; 6.8291x over previous
import functools

import jax
import jax.numpy as jnp
from jax import lax
from jax.experimental import pallas as pl
from jax.experimental.pallas import tpu as pltpu
from jax.experimental.pallas import tpu_sc as plsc

N = 10000
E = 320000
D = 128
NC = 2
NS = 16
L = 16

EPT = E // (NC * NS)
K = 80
NCHUNK = EPT // K
RPT = 624
RB = 104

_GDN = lax.GatherDimensionNumbers(
    offset_dims=(), collapsed_slice_dims=(0,), start_index_map=(0,))


def _bcast_lane(vec, i):
    idx = jnp.full((L, 1), i, jnp.int32)
    return lax.gather(vec, idx, dimension_numbers=_GDN, slice_sizes=(1,),
                      mode=lax.GatherScatterMode.PROMISE_IN_BOUNDS)


def _sc_body(h, src, dst3, ew, p0, p1, acc, src_v, dst_v, ew_v,
             rows_v, sem):
    c = lax.axis_index("c")
    s = lax.axis_index("s")
    w = c * NS + s
    ebase = w * EPT

    pltpu.sync_copy(src.at[pl.ds(ebase, EPT)], src_v)
    pltpu.sync_copy(dst3.at[w], dst_v)
    pltpu.sync_copy(ew.at[pl.ds(ebase, EPT)], ew_v)

    zvec = jnp.zeros((L,), jnp.float32)

    def z_body(i, carry):
        for k in range(D // L):
            rows_v[i, pl.ds(k * L, L)] = zvec
        return carry

    lax.fori_loop(0, K, z_body, 0, unroll=4)
    rbase = s * RPT
    rem = N - NS * RPT
    for j in range(RPT // K):
        pltpu.sync_copy(rows_v, acc.at[pl.ds(rbase + j * K, K)])
    pltpu.sync_copy(rows_v.at[pl.ds(0, RPT - (RPT // K) * K)],
                    acc.at[pl.ds(rbase + (RPT // K) * K,
                                 RPT - (RPT // K) * K)])

    @pl.when(s == NS - 1)
    def _():
        pltpu.sync_copy(rows_v.at[pl.ds(0, rem)],
                        acc.at[pl.ds(NS * RPT, rem)])

    plsc.subcore_barrier()

    def chunk_body(t, carry):
        eb = t * K
        pltpu.async_copy(h.at[src_v.at[pl.ds(eb, K)]], rows_v, sem).wait()

        def grp_body(g, carry2):
            ewv = ew_v[pl.ds(eb + g * L, L)]
            r0 = g * L
            for i in range(L):
                wv = _bcast_lane(ewv, i)
                for k in range(D // L):
                    sl = pl.ds(k * L, L)
                    rows_v[r0 + i, sl] = rows_v[r0 + i, sl] * wv
            return carry2

        lax.fori_loop(0, K // L, grp_body, 0)
        pltpu.sync_copy(rows_v, acc.at[dst_v.at[t]], add=True)
        return carry

    lax.fori_loop(0, NCHUNK, chunk_body, 0)
    plsc.subcore_barrier()

    @pl.when(c == 0)
    def _():
        pltpu.sync_copy(acc.at[pl.ds(rbase, RPT)], p0.at[pl.ds(rbase, RPT)])

        @pl.when(s == NS - 1)
        def _():
            pltpu.sync_copy(acc.at[pl.ds(NS * RPT, rem)],
                            p0.at[pl.ds(NS * RPT, rem)])

    @pl.when(c == 1)
    def _():
        pltpu.sync_copy(acc.at[pl.ds(rbase, RPT)], p1.at[pl.ds(rbase, RPT)])

        @pl.when(s == NS - 1)
        def _():
            pltpu.sync_copy(acc.at[pl.ds(NS * RPT, rem)],
                            p1.at[pl.ds(NS * RPT, rem)])


_sc_aggregate = functools.partial(
    pl.kernel,
    out_type=(jax.ShapeDtypeStruct((N, D), jnp.float32),
              jax.ShapeDtypeStruct((N, D), jnp.float32)),
    mesh=plsc.VectorSubcoreMesh(core_axis_name="c", subcore_axis_name="s"),
    scratch_types=[
        pltpu.VMEM_SHARED((N, D), jnp.float32),
        pltpu.VMEM((EPT,), jnp.int32),
        pltpu.VMEM((NCHUNK, K), jnp.int32),
        pltpu.VMEM((EPT,), jnp.float32),
        pltpu.VMEM((K, D), jnp.float32),
        pltpu.SemaphoreType.DMA,
    ],
)(_sc_body)


def _mm_body(p0_ref, p1_ref, w_ref, o_ref):
    agg = p0_ref[...] + p1_ref[...]
    acc = jnp.dot(agg, w_ref[...], preferred_element_type=jnp.float32)
    o_ref[...] = jnp.maximum(acc, 0.0)


def _matmul_relu(p0, p1, weight):
    grid = 10
    rb = N // grid
    return pl.pallas_call(
        _mm_body,
        grid=(grid,),
        in_specs=[
            pl.BlockSpec((rb, D), lambda i: (i, 0)),
            pl.BlockSpec((rb, D), lambda i: (i, 0)),
            pl.BlockSpec((D, D), lambda i: (0, 0)),
        ],
        out_specs=pl.BlockSpec((rb, D), lambda i: (i, 0)),
        out_shape=jax.ShapeDtypeStruct((N, D), jnp.float32),
    )(p0, p1, weight)


@jax.jit
def kernel(h, edge_index, edge_weight, weight):
    src = edge_index[0].astype(jnp.int32)
    dst3 = edge_index[1].astype(jnp.int32).reshape(NC * NS, NCHUNK, K)
    p0, p1 = _sc_aggregate(h, src, dst3, edge_weight)
    return _matmul_relu(p0, p1, weight)
